# baseline (speedup 1.0000x reference)
import jax
import jax.numpy as jnp
from jax.experimental import pallas as pl
from jax.experimental.pallas import tpu as pltpu
from jax.experimental.pallas import tpu_sc as plsc

_B, _L, _DIM, _DQK = 4, 8192, 768, 128
_BLK = 512
_NB = _L // _BLK
_GW = 128


def _shift_down(arr, d, fill):
    row = jax.lax.broadcasted_iota(jnp.int32, arr.shape, 0)
    rolled = jnp.roll(arr, d, axis=0)
    return jnp.where(row < d, fill, rolled)


def _tc_body(tok_ref, w_ref, sk_ref, h_ref, src_ref, key_c, h_c, scal_c):
    b = pl.program_id(0)
    i = pl.program_id(1)

    @pl.when(i == 0)
    def _init():
        key_c[...] = jnp.broadcast_to(sk_ref[...], key_c.shape)
        h_c[...] = jnp.zeros_like(h_c)
        scal_c[0] = 0
        scal_c[1] = -1

    tok = tok_ref[0]
    qk = jnp.dot(tok, w_ref[...], preferred_element_type=jnp.float32)
    q = qk[:, :_DQK]
    k = qk[:, _DQK:]
    kprev = _shift_down(k, 1, 0.0)
    row = jax.lax.broadcasted_iota(jnp.int32, (_BLK, 1), 0)
    kprev = jnp.where(row == 0, key_c[0:1, :_DQK], kprev)

    nq = jnp.maximum(jnp.sqrt(jnp.sum(q * q, axis=1, keepdims=True)), 1e-8)
    nk = jnp.maximum(jnp.sqrt(jnp.sum(kprev * kprev, axis=1, keepdims=True)), 1e-8)
    cos = jnp.sum(q * kprev, axis=1, keepdims=True) / (nq * nk)
    probs = (1.0 - cos) * 0.5

    bnd = probs > 0.5
    bnd = jnp.logical_or(bnd, jnp.logical_and(i == 0, row == 0))

    g = jnp.where(bnd, 1.0 - probs, 1.0)
    x = jnp.where(bnd, probs, 0.0) * tok

    A, S = g, x
    cnt = bnd.astype(jnp.int32)
    d = 1
    while d < _BLK:
        S = A * _shift_down(S, d, 0.0) + S
        cnt = cnt + _shift_down(cnt, d, 0)
        A = A * _shift_down(A, d, 1.0)
        d *= 2

    h = S + A * h_c[0:1, :]
    h_ref[0] = h

    key_c[0:1, :_DQK] = k[_BLK - 1:_BLK, :]
    h_c[0:1, :] = h[_BLK - 1:_BLK, :]

    rank = cnt - 1
    j_iota = jax.lax.broadcasted_iota(jnp.int32, (1, _BLK), 1)
    M = jnp.logical_and(bnd, rank == j_iota).astype(jnp.float32)
    posf = b * _L + i * _BLK + j_iota
    hi = (posf >> 8).astype(jnp.float32)
    lo = (posf & 255).astype(jnp.float32)
    stacked = jnp.concatenate([hi, lo], axis=0)
    packed2 = jnp.dot(stacked, M, preferred_element_type=jnp.float32)
    packed = (packed2[0:1] * 256.0 + packed2[1:2] + 0.5).astype(jnp.int32)

    cnt_prev = scal_c[0]
    lane = jax.lax.broadcasted_iota(jnp.int32, (1, _L), 1)
    padded = jnp.pad(packed, ((0, 0), (0, _L - _BLK)))
    rotated = pltpu.roll(padded, cnt_prev, axis=1)
    in_window = jnp.logical_and(lane >= cnt_prev, lane < cnt_prev + _BLK)
    src_ref[0] = jnp.where(in_window, rotated, src_ref[0])

    pos_col = b * _L + i * _BLK + row
    blk_cnt = jnp.sum(bnd.astype(jnp.int32))
    last_blk = jnp.max(jnp.where(bnd, pos_col, -1))
    scal_c[0] = cnt_prev + blk_cnt
    scal_c[1] = jnp.maximum(scal_c[1], last_blk)

    @pl.when(i == _NB - 1)
    def _tail():
        kk = scal_c[0]
        last = scal_c[1]
        lane = jax.lax.broadcasted_iota(jnp.int32, (1, _L), 1)
        src_ref[0] = jnp.where(lane >= kk, last, src_ref[0])


def _tc_scan(tokens, W_qk, start_key, interpret=False):
    sk = start_key.reshape(1, _DQK)
    grid = (_B, _NB)
    h, src = pl.pallas_call(
        _tc_body,
        grid=grid,
        in_specs=[
            pl.BlockSpec((1, _BLK, _DIM), lambda b, i: (b, i, 0)),
            pl.BlockSpec((_DIM, 2 * _DQK), lambda b, i: (0, 0)),
            pl.BlockSpec((1, _DQK), lambda b, i: (0, 0)),
        ],
        out_specs=[
            pl.BlockSpec((1, _BLK, _DIM), lambda b, i: (b, i, 0)),
            pl.BlockSpec((1, 1, _L), lambda b, i: (b, 0, 0)),
        ],
        out_shape=[
            jax.ShapeDtypeStruct((_B, _L, _DIM), jnp.float32),
            jax.ShapeDtypeStruct((_B, 1, _L), jnp.int32),
        ],
        scratch_shapes=[
            pltpu.VMEM((8, _DQK), jnp.float32),
            pltpu.VMEM((8, _DIM), jnp.float32),
            pltpu.SMEM((4,), jnp.int32),
        ],
        interpret=interpret,
    )(tokens, W_qk, sk)
    return h, src.reshape(1, _B * _L)


_NW = 32
_RPW = (_B * _L) // _NW
_CH = 64
_NCH = _RPW // _CH


def _sc_gather(h_flat, src):
    n = _B * _L
    mesh = plsc.VectorSubcoreMesh(core_axis_name="c", subcore_axis_name="s")

    @pl.kernel(
        out_type=jax.ShapeDtypeStruct((n, _DIM), jnp.float32),
        scratch_types=[
            pltpu.VMEM((_NCH, _CH), jnp.int32),
            pltpu.VMEM((_CH, _DIM), jnp.float32),
            pltpu.VMEM((_CH, _DIM), jnp.float32),
            pltpu.SemaphoreType.DMA,
            pltpu.SemaphoreType.DMA,
            pltpu.SemaphoreType.DMA,
            pltpu.SemaphoreType.DMA,
        ],
        mesh=mesh,
    )
    def k(h_hbm, i_hbm, o_hbm, idx_all, rows0, rows1, sg0, sg1, sw0, sw1):
        wid = jax.lax.axis_index("s") * 2 + jax.lax.axis_index("c")
        base = wid * _RPW
        pltpu.sync_copy(i_hbm.at[pl.ds(wid * _NCH, _NCH)], idx_all)

        def gather(c, rows_v, sg):
            pltpu.async_copy(h_hbm.at[idx_all.at[c]], rows_v, sg)

        def wait_gather(rows_v, sg):
            pltpu.make_async_copy(
                h_hbm.at[idx_all.at[0]], rows_v, sg).wait()

        def wait_wb(rows_v, sw):
            pltpu.make_async_copy(
                rows_v, o_hbm.at[pl.ds(base, _CH)], sw).wait()

        gather(0, rows0, sg0)

        def halfstep(c, rows_p, sg_p, sw_p, rows_q, sg_q, sw_q):
            @pl.when(c + 1 < _NCH)
            def _issue():
                @pl.when(c >= 1)
                def _wb_done():
                    wait_wb(rows_q, sw_q)

                gather(c + 1, rows_q, sg_q)

            wait_gather(rows_p, sg_p)
            pltpu.async_copy(rows_p, o_hbm.at[pl.ds(base + c * _CH, _CH)], sw_p)

        def body(c, carry):
            @pl.when(c % 2 == 0)
            def _even():
                halfstep(c, rows0, sg0, sw0, rows1, sg1, sw1)

            @pl.when(c % 2 == 1)
            def _odd():
                halfstep(c, rows1, sg1, sw1, rows0, sg0, sw0)

            return carry

        jax.lax.fori_loop(0, _NCH, body, 0)
        wait_wb(rows0, sw0)
        wait_wb(rows1, sw1)

    return k(h_flat, src)


def kernel(tokens, W_qk, start_key):
    h, src = _tc_scan(tokens, W_qk, start_key)
    smoothed = _sc_gather(h.reshape(_B * _L, _DIM),
                          src.reshape(_NW * _NCH, _CH))
    smoothed = smoothed.reshape(_B, _L, _DIM)
    aux = jnp.zeros((), jnp.float32)
    return smoothed, h, aux


# speedup vs baseline: 1.9700x; 1.9700x over previous
import jax
import jax.numpy as jnp
from jax.experimental import pallas as pl
from jax.experimental.pallas import tpu as pltpu
from jax.experimental.pallas import tpu_sc as plsc

_B, _L, _DIM, _DQK = 4, 8192, 768, 128
_BLK = 512
_NB = _L // _BLK
_GW = 128


def _shift_down(arr, d, fill):
    row = jax.lax.broadcasted_iota(jnp.int32, arr.shape, 0)
    rolled = jnp.roll(arr, d, axis=0)
    return jnp.where(row < d, fill, rolled)


def _tc_body(tok_ref, w_ref, sk_ref, h_ref, src_ref, key_c, h_c, scal_c):
    b = pl.program_id(0)
    i = pl.program_id(1)

    @pl.when(i == 0)
    def _init():
        key_c[...] = jnp.broadcast_to(sk_ref[...], key_c.shape)
        h_c[...] = jnp.zeros_like(h_c)
        scal_c[0] = 0
        scal_c[1] = -1

    tok = tok_ref[0]
    qk = jnp.dot(tok, w_ref[...], preferred_element_type=jnp.float32)
    q = qk[:, :_DQK]
    k = qk[:, _DQK:]
    kprev = _shift_down(k, 1, 0.0)
    row = jax.lax.broadcasted_iota(jnp.int32, (_BLK, 1), 0)
    kprev = jnp.where(row == 0, key_c[0:1, :_DQK], kprev)

    nq = jnp.maximum(jnp.sqrt(jnp.sum(q * q, axis=1, keepdims=True)), 1e-8)
    nk = jnp.maximum(jnp.sqrt(jnp.sum(kprev * kprev, axis=1, keepdims=True)), 1e-8)
    cos = jnp.sum(q * kprev, axis=1, keepdims=True) / (nq * nk)
    probs = (1.0 - cos) * 0.5

    bnd = probs > 0.5
    bnd = jnp.logical_or(bnd, jnp.logical_and(i == 0, row == 0))

    g = jnp.where(bnd, 1.0 - probs, 1.0)
    x = jnp.where(bnd, probs, 0.0) * tok

    A, S = g, x
    cnt = bnd.astype(jnp.int32)
    d = 1
    while d < _BLK:
        S = A * _shift_down(S, d, 0.0) + S
        cnt = cnt + _shift_down(cnt, d, 0)
        A = A * _shift_down(A, d, 1.0)
        d *= 2

    h = S + A * h_c[0:1, :]
    h_ref[0] = h

    key_c[0:1, :_DQK] = k[_BLK - 1:_BLK, :]
    h_c[0:1, :] = h[_BLK - 1:_BLK, :]

    rank = cnt - 1
    j_iota = jax.lax.broadcasted_iota(jnp.int32, (1, _BLK), 1)
    M = jnp.logical_and(bnd, rank == j_iota).astype(jnp.float32)
    posf = b * _L + i * _BLK + j_iota
    hi = (posf >> 8).astype(jnp.float32)
    lo = (posf & 255).astype(jnp.float32)
    stacked = jnp.concatenate([hi, lo], axis=0)
    packed2 = jnp.dot(stacked, M, preferred_element_type=jnp.float32)
    packed = (packed2[0:1] * 256.0 + packed2[1:2] + 0.5).astype(jnp.int32)

    cnt_prev = scal_c[0]
    lane = jax.lax.broadcasted_iota(jnp.int32, (1, _L), 1)
    padded = jnp.pad(packed, ((0, 0), (0, _L - _BLK)))
    rotated = pltpu.roll(padded, cnt_prev, axis=1)
    in_window = jnp.logical_and(lane >= cnt_prev, lane < cnt_prev + _BLK)
    src_ref[0] = jnp.where(in_window, rotated, src_ref[0])

    pos_col = b * _L + i * _BLK + row
    blk_cnt = jnp.sum(bnd.astype(jnp.int32))
    last_blk = jnp.max(jnp.where(bnd, pos_col, -1))
    scal_c[0] = cnt_prev + blk_cnt
    scal_c[1] = jnp.maximum(scal_c[1], last_blk)

    @pl.when(i == _NB - 1)
    def _tail():
        kk = scal_c[0]
        last = scal_c[1]
        lane = jax.lax.broadcasted_iota(jnp.int32, (1, _L), 1)
        src_ref[0] = jnp.where(lane >= kk, last, src_ref[0])


def _tc_scan(tokens, W_qk, start_key, interpret=False):
    sk = start_key.reshape(1, _DQK)
    grid = (_B, _NB)
    h, src = pl.pallas_call(
        _tc_body,
        grid=grid,
        in_specs=[
            pl.BlockSpec((1, _BLK, _DIM), lambda b, i: (b, i, 0)),
            pl.BlockSpec((_DIM, 2 * _DQK), lambda b, i: (0, 0)),
            pl.BlockSpec((1, _DQK), lambda b, i: (0, 0)),
        ],
        out_specs=[
            pl.BlockSpec((1, _BLK, _DIM), lambda b, i: (b, i, 0)),
            pl.BlockSpec((1, 1, _L), lambda b, i: (b, 0, 0)),
        ],
        out_shape=[
            jax.ShapeDtypeStruct((_B, _L, _DIM), jnp.float32),
            jax.ShapeDtypeStruct((_B, 1, _L), jnp.int32),
        ],
        scratch_shapes=[
            pltpu.VMEM((8, _DQK), jnp.float32),
            pltpu.VMEM((8, _DIM), jnp.float32),
            pltpu.SMEM((4,), jnp.int32),
        ],
        interpret=interpret,
    )(tokens, W_qk, sk)
    return h, src.reshape(1, _B * _L)


_NW = 32
_RPW = (_B * _L) // _NW
_CH = 64
_NCH = _RPW // _CH


def _sc_gather(h_flat, src):
    n = _B * _L
    mesh = plsc.VectorSubcoreMesh(core_axis_name="c", subcore_axis_name="s")

    @pl.kernel(
        out_type=jax.ShapeDtypeStruct((n, _DIM), jnp.float32),
        scratch_types=[
            pltpu.VMEM((_NCH, _CH), jnp.int32),
            pltpu.VMEM((_CH, _DIM), jnp.float32),
            pltpu.SemaphoreType.DMA,
            pltpu.SemaphoreType.DMA,
        ],
        mesh=mesh,
    )
    def k(h_hbm, i_hbm, o_hbm, idx_all, rows_v, sg, sw):
        wid = jax.lax.axis_index("s") * 2 + jax.lax.axis_index("c")
        base = wid * _RPW
        pltpu.sync_copy(i_hbm.at[pl.ds(wid * _NCH, _NCH)], idx_all)

        def wait_wb(j, cc):
            pltpu.make_async_copy(
                rows_v, o_hbm.at[pl.ds(base, _CH)], sw).wait()
            return cc

        def body(c, carry):
            prev, nout = carry
            v0 = idx_all[c, pl.ds(0, 16)]
            v1 = idx_all[c, pl.ds(_CH - 16, 16)]
            lo = v0[0]
            hi = v1[15]
            is_const = lo == hi
            is_rep = jnp.logical_and(is_const, hi == prev)

            @pl.when(jnp.logical_not(is_rep))
            def _gather():
                jax.lax.fori_loop(0, nout, wait_wb, 0)
                pltpu.async_copy(h_hbm.at[idx_all.at[c]], rows_v, sg).wait()

            pltpu.async_copy(
                rows_v, o_hbm.at[pl.ds(base + c * _CH, _CH)], sw)
            nout2 = jnp.where(is_rep, nout + 1, 1)
            prev2 = jnp.where(is_const, hi, -1)
            return (prev2, nout2)

        _, nout = jax.lax.fori_loop(0, _NCH, body, (-1, 0))
        jax.lax.fori_loop(0, nout, wait_wb, 0)

    return k(h_flat, src)


def kernel(tokens, W_qk, start_key):
    h, src = _tc_scan(tokens, W_qk, start_key)
    smoothed = _sc_gather(h.reshape(_B * _L, _DIM),
                          src.reshape(_NW * _NCH, _CH))
    smoothed = smoothed.reshape(_B, _L, _DIM)
    aux = jnp.zeros((), jnp.float32)
    return smoothed, h, aux
